# Initial kernel scaffold; baseline (speedup 1.0000x reference)
import jax
import jax.numpy as jnp
from jax.experimental import pallas as pl


def kernel(pred_heatmap, gt_keypoints):
    raise NotImplementedError("write your pallas kernel here")


# speedup vs baseline: 42.8105x; 42.8105x over previous
import functools

import jax
import jax.numpy as jnp
from jax import lax
from jax.experimental import pallas as pl
from jax.experimental.pallas import tpu as pltpu
from jax.experimental.pallas import tpu_sc as plsc

_B, _K, _H, _W = 4, 17, 512, 512
_ROWS = _B * _K
_NPIX = _H * _W
_KEEP = int(_NPIX * 0.2)
_NBINS = 16384
_SHIFT = 17
_PAD_ROWS = 72
_NW = 32
_CHUNK_IMG_ROWS = 64
_CHUNK = _CHUNK_IMG_ROWS * _W
_NCHUNK = _NPIX // _CHUNK
_RWIN = 24.0
_INV2S2 = 1.0 / 32.0


def _hist_body(pred_hbm, kp_hbm, cnt_hbm, sum_hbm, buf, cnt_v, sum_v, kp_v):
    wid = lax.axis_index("s") * 2 + lax.axis_index("c")

    pltpu.sync_copy(kp_hbm, kp_v)

    ones16 = jnp.ones((16,), jnp.float32)

    def accumulate(loss):
        bits = lax.bitcast_convert_type(loss, jnp.int32)
        bins = lax.shift_right_logical(bits, _SHIFT)
        plsc.addupdate_scatter(cnt_v, [bins], ones16)
        plsc.addupdate_scatter(sum_v, [bins], loss)

    def process_row(r):
        def zbody(i, carry):
            z = jnp.zeros((16,), jnp.float32)
            cnt_v[pl.ds(i * 16, 16)] = z
            sum_v[pl.ds(i * 16, 16)] = z
            return carry
        lax.fori_loop(0, _NBINS // 16, zbody, 0)

        kvec = kp_v[pl.ds(r * 16, 16)]
        kx = kvec[0]
        ky = kvec[1]

        def chunk_body(c, carry):
            pltpu.sync_copy(pred_hbm.at[pl.ds(r * _NPIX + c * _CHUNK, _CHUNK)], buf)
            ybase = c * _CHUNK_IMG_ROWS

            def yrow_body(yy, ycarry):
                yf = (ybase + yy).astype(jnp.float32)
                dy = yf - ky
                in_win = jnp.abs(dy) <= _RWIN

                @pl.when(jnp.logical_not(in_win))
                def _():
                    def jb(j, jcarry):
                        v = buf[pl.ds(yy * _W + j * 16, 16)]
                        accumulate(v * v)
                        return jcarry
                    lax.fori_loop(0, _W // 16, jb, 0)

                @pl.when(in_win)
                def _():
                    dy2 = dy * dy

                    def jb(j, jcarry):
                        v = buf[pl.ds(yy * _W + j * 16, 16)]
                        xv = (lax.iota(jnp.int32, 16) + j * 16).astype(jnp.float32)
                        dx = xv - kx
                        d2 = dx * dx + dy2
                        gt = jnp.exp(d2 * (-_INV2S2))
                        dlt = v - gt
                        accumulate(dlt * dlt)
                        return jcarry
                    lax.fori_loop(0, _W // 16, jb, 0)

                return ycarry
            lax.fori_loop(0, _CHUNK_IMG_ROWS, yrow_body, 0)
            return carry
        lax.fori_loop(0, _NCHUNK, chunk_body, 0)

        pltpu.sync_copy(cnt_v, cnt_hbm.at[r])
        pltpu.sync_copy(sum_v, sum_hbm.at[r])

    process_row(wid)
    process_row(wid + 32)

    @pl.when(wid < _ROWS - 64)
    def _():
        process_row(wid + 64)


@functools.cache
def _hist():
    return pl.kernel(
        _hist_body,
        out_type=(
            jax.ShapeDtypeStruct((_PAD_ROWS, _NBINS), jnp.float32),
            jax.ShapeDtypeStruct((_PAD_ROWS, _NBINS), jnp.float32),
        ),
        mesh=plsc.VectorSubcoreMesh(
            core_axis_name="c", subcore_axis_name="s", num_cores=2, num_subcores=16
        ),
        scratch_types=[
            pltpu.VMEM((_CHUNK,), jnp.float32),
            pltpu.VMEM((_NBINS,), jnp.float32),
            pltpu.VMEM((_NBINS,), jnp.float32),
            pltpu.VMEM((16 * _ROWS,), jnp.float32),
        ],
        compiler_params=pltpu.CompilerParams(needs_layout_passes=False),
    )


def _select_body(cnt_ref, sum_ref, out_ref):
    cnt = cnt_ref[...]
    sm = sum_ref[...]

    i2 = lax.broadcasted_iota(jnp.int32, (128, 128), 0)
    j2 = lax.broadcasted_iota(jnp.int32, (128, 128), 1)
    t_incl = (i2 <= j2).astype(jnp.float32)
    t_strict = (i2 < j2).astype(jnp.float32)

    dn3 = (((2,), (0,)), ((), ()))
    fine_c = lax.dot_general(cnt, t_incl, dn3, preferred_element_type=jnp.float32)
    fine_s = lax.dot_general(sm, t_incl, dn3, preferred_element_type=jnp.float32)

    cs_c = jnp.sum(cnt, axis=2)
    cs_s = jnp.sum(sm, axis=2)
    dn2 = (((1,), (0,)), ((), ()))
    coarse_c = lax.dot_general(cs_c, t_strict, dn2, preferred_element_type=jnp.float32)
    coarse_s = lax.dot_general(cs_s, t_strict, dn2, preferred_element_type=jnp.float32)

    cum_c = coarse_c[:, :, None] + fine_c
    cum_s = coarse_s[:, :, None] + fine_s

    bi = (lax.broadcasted_iota(jnp.int32, (_PAD_ROWS, 128, 128), 1) * 128
          + lax.broadcasted_iota(jnp.int32, (_PAD_ROWS, 128, 128), 2)
          ).astype(jnp.float32)
    crossed = cum_c > float(_NPIX - _KEEP)
    bstar = jnp.min(jnp.where(crossed, bi, jnp.float32(3.0e7)), axis=2)
    bstar = jnp.min(bstar, axis=1, keepdims=True)
    sel = (bi == bstar[:, :, None]).astype(jnp.float32)

    def pick(x):
        return jnp.sum(jnp.sum(sel * x, axis=2), axis=1, keepdims=True)

    cum_c_b = pick(cum_c)
    cum_s_b = pick(cum_s)
    cnt_b = pick(cnt)
    sum_b = pick(sm)
    s_tot = jnp.sum(jnp.sum(sm, axis=2), axis=1, keepdims=True)

    need = float(_KEEP - _NPIX) + cum_c_b
    est = sum_b / jnp.maximum(cnt_b, 1.0)
    row_sum = (s_tot - cum_s_b) + need * est

    valid = lax.broadcasted_iota(jnp.int32, (_PAD_ROWS, 1), 0) < _ROWS
    total = jnp.sum(jnp.where(valid, row_sum, 0.0))
    out_ref[...] = jnp.broadcast_to(total / float(_ROWS * _KEEP), (1, 1))


_select = pl.pallas_call(
    _select_body,
    out_shape=jax.ShapeDtypeStruct((1, 1), jnp.float32),
)


def kernel(pred_heatmap, gt_keypoints):
    pred_flat = pred_heatmap.reshape(-1)
    kp_pad = jnp.zeros((_ROWS, 16), jnp.float32)
    kp_flat = kp_pad.at[:, :2].set(gt_keypoints.reshape(_ROWS, 2)).reshape(-1)
    cnt, sm = _hist()(pred_flat, kp_flat)
    out = _select(cnt.reshape(_PAD_ROWS, 128, 128), sm.reshape(_PAD_ROWS, 128, 128))
    return out.reshape(())


# speedup vs baseline: 48.4744x; 1.1323x over previous
import functools

import jax
import jax.numpy as jnp
from jax import lax
from jax.experimental import pallas as pl
from jax.experimental.pallas import tpu as pltpu
from jax.experimental.pallas import tpu_sc as plsc

_B, _K, _H, _W = 4, 17, 512, 512
_ROWS = _B * _K
_NPIX = _H * _W
_KEEP = int(_NPIX * 0.2)
_NBINS = 16384
_SHIFT = 17
_PAD_ROWS = 72
_NW = 32
_CHUNK_IMG_ROWS = 64
_CHUNK = _CHUNK_IMG_ROWS * _W
_NCHUNK = _NPIX // _CHUNK
_RWIN = 24.0
_INV2S2 = 1.0 / 32.0


def _hist_body(pred_hbm, kp_hbm, cnt_hbm, sum_hbm, buf0, buf1, cnt_v, sum_v,
               kp_v, sem0, sem1):
    wid = lax.axis_index("s") * 2 + lax.axis_index("c")

    pltpu.sync_copy(kp_hbm, kp_v)

    ones16 = jnp.ones((16,), jnp.float32)

    def accumulate(loss):
        bits = lax.bitcast_convert_type(loss, jnp.int32)
        bins = lax.shift_right_logical(bits, _SHIFT)
        plsc.addupdate_scatter(cnt_v, [bins], ones16)
        plsc.addupdate_scatter(sum_v, [bins], loss)

    def chunk_src(r, cc):
        return pred_hbm.at[pl.ds(r * _NPIX + cc * _CHUNK, _CHUNK)]

    def process_row(r):
        def zbody(i, carry):
            z = jnp.zeros((16,), jnp.float32)
            for u in range(4):
                cnt_v[pl.ds(i * 64 + u * 16, 16)] = z
                sum_v[pl.ds(i * 64 + u * 16, 16)] = z
            return carry
        lax.fori_loop(0, _NBINS // 64, zbody, 0)

        kvec = kp_v[pl.ds(r * 16, 16)]
        kx = kvec[0]
        ky = kvec[1]
        kyi = ky.astype(jnp.int32)
        wlo = jnp.clip(kyi - 26, 0, _H)
        whi = jnp.clip(kyi + 27, 0, _H)
        wstart = wlo * _W
        wend = whi * _W

        def pure_range(buf, lo, hi):
            def body(i, carry):
                base = lo + i * 64
                for u in range(4):
                    v = buf[pl.ds(base + u * 16, 16)]
                    accumulate(v * v)
                return carry
            lax.fori_loop(0, (hi - lo) // 64, body, 0)

        def win_range(buf, lo, hi, a):
            def wbody(iy, carry):
                off = lo + iy * _W
                y = lax.shift_right_logical(a + off, 9)
                dy = y.astype(jnp.float32) - ky
                dy2 = dy * dy

                def jb(j, c2):
                    v = buf[pl.ds(off + j * 16, 16)]
                    xv = (lax.iota(jnp.int32, 16) + j * 16).astype(jnp.float32)
                    dx = xv - kx
                    gt = jnp.exp((dx * dx + dy2) * (-_INV2S2))
                    dlt = v - gt
                    accumulate(dlt * dlt)
                    return c2
                lax.fori_loop(0, _W // 16, jb, 0)
                return carry
            lax.fori_loop(0, (hi - lo) // _W, wbody, 0)

        def process_chunk(buf, cc):
            a = cc * _CHUNK
            p1 = jnp.clip(wstart, a, a + _CHUNK) - a
            p2 = jnp.clip(wend, a, a + _CHUNK) - a
            pure_range(buf, 0, p1)
            win_range(buf, p1, p2, a)
            pure_range(buf, p2, _CHUNK)

        pltpu.async_copy(chunk_src(r, 0), buf0, sem0)

        def hbody(h, carry):
            c0 = 2 * h
            pltpu.async_copy(chunk_src(r, c0 + 1), buf1, sem1)
            pltpu.make_async_copy(chunk_src(r, c0), buf0, sem0).wait()
            process_chunk(buf0, c0)

            @pl.when(h < _NCHUNK // 2 - 1)
            def _():
                pltpu.async_copy(chunk_src(r, c0 + 2), buf0, sem0)

            pltpu.make_async_copy(chunk_src(r, c0 + 1), buf1, sem1).wait()
            process_chunk(buf1, c0 + 1)
            return carry
        lax.fori_loop(0, _NCHUNK // 2, hbody, 0)

        pltpu.sync_copy(cnt_v, cnt_hbm.at[r])
        pltpu.sync_copy(sum_v, sum_hbm.at[r])

    nrows = jnp.where(wid < _ROWS - 64, 3, 2)

    def row_body(i, carry):
        process_row(wid + 32 * i)
        return carry
    lax.fori_loop(0, nrows, row_body, 0)


@functools.cache
def _hist():
    return pl.kernel(
        _hist_body,
        out_type=(
            jax.ShapeDtypeStruct((_PAD_ROWS, _NBINS), jnp.float32),
            jax.ShapeDtypeStruct((_PAD_ROWS, _NBINS), jnp.float32),
        ),
        mesh=plsc.VectorSubcoreMesh(
            core_axis_name="c", subcore_axis_name="s", num_cores=2, num_subcores=16
        ),
        scratch_types=[
            pltpu.VMEM((_CHUNK,), jnp.float32),
            pltpu.VMEM((_CHUNK,), jnp.float32),
            pltpu.VMEM((_NBINS,), jnp.float32),
            pltpu.VMEM((_NBINS,), jnp.float32),
            pltpu.VMEM((16 * _ROWS,), jnp.float32),
            pltpu.SemaphoreType.DMA,
            pltpu.SemaphoreType.DMA,
        ],
        compiler_params=pltpu.CompilerParams(needs_layout_passes=False),
    )


def _select_body(cnt_ref, sum_ref, out_ref):
    cnt = cnt_ref[...]
    sm = sum_ref[...]

    i2 = lax.broadcasted_iota(jnp.int32, (128, 128), 0)
    j2 = lax.broadcasted_iota(jnp.int32, (128, 128), 1)
    t_incl = (i2 <= j2).astype(jnp.float32)
    t_strict = (i2 < j2).astype(jnp.float32)

    dn3 = (((2,), (0,)), ((), ()))
    fine_c = lax.dot_general(cnt, t_incl, dn3, preferred_element_type=jnp.float32)
    fine_s = lax.dot_general(sm, t_incl, dn3, preferred_element_type=jnp.float32)

    cs_c = jnp.sum(cnt, axis=2)
    cs_s = jnp.sum(sm, axis=2)
    dn2 = (((1,), (0,)), ((), ()))
    coarse_c = lax.dot_general(cs_c, t_strict, dn2, preferred_element_type=jnp.float32)
    coarse_s = lax.dot_general(cs_s, t_strict, dn2, preferred_element_type=jnp.float32)

    cum_c = coarse_c[:, :, None] + fine_c
    cum_s = coarse_s[:, :, None] + fine_s

    bi = (lax.broadcasted_iota(jnp.int32, (_PAD_ROWS, 128, 128), 1) * 128
          + lax.broadcasted_iota(jnp.int32, (_PAD_ROWS, 128, 128), 2)
          ).astype(jnp.float32)
    crossed = cum_c > float(_NPIX - _KEEP)
    bstar = jnp.min(jnp.where(crossed, bi, jnp.float32(3.0e7)), axis=2)
    bstar = jnp.min(bstar, axis=1, keepdims=True)
    sel = (bi == bstar[:, :, None]).astype(jnp.float32)

    def pick(x):
        return jnp.sum(jnp.sum(sel * x, axis=2), axis=1, keepdims=True)

    cum_c_b = pick(cum_c)
    cum_s_b = pick(cum_s)
    cnt_b = pick(cnt)
    sum_b = pick(sm)
    s_tot = jnp.sum(jnp.sum(sm, axis=2), axis=1, keepdims=True)

    need = float(_KEEP - _NPIX) + cum_c_b
    est = sum_b / jnp.maximum(cnt_b, 1.0)
    row_sum = (s_tot - cum_s_b) + need * est

    valid = lax.broadcasted_iota(jnp.int32, (_PAD_ROWS, 1), 0) < _ROWS
    total = jnp.sum(jnp.where(valid, row_sum, 0.0))
    out_ref[...] = jnp.broadcast_to(total / float(_ROWS * _KEEP), (1, 1))


_select = pl.pallas_call(
    _select_body,
    out_shape=jax.ShapeDtypeStruct((1, 1), jnp.float32),
)


def kernel(pred_heatmap, gt_keypoints):
    pred_flat = pred_heatmap.reshape(-1)
    kp_pad = jnp.zeros((_ROWS, 16), jnp.float32)
    kp_flat = kp_pad.at[:, :2].set(gt_keypoints.reshape(_ROWS, 2)).reshape(-1)
    cnt, sm = _hist()(pred_flat, kp_flat)
    out = _select(cnt.reshape(_PAD_ROWS, 128, 128), sm.reshape(_PAD_ROWS, 128, 128))
    return out.reshape(())


# speedup vs baseline: 54.3570x; 1.1214x over previous
import functools

import jax
import jax.numpy as jnp
from jax import lax
from jax.experimental import pallas as pl
from jax.experimental.pallas import tpu as pltpu
from jax.experimental.pallas import tpu_sc as plsc

_B, _K, _H, _W = 4, 17, 512, 512
_ROWS = _B * _K
_NPIX = _H * _W
_KEEP = int(_NPIX * 0.2)
_NBINS = 16384
_SHIFT = 17
_PAD_ROWS = 72
_NW = 32
_CHUNK_IMG_ROWS = 64
_CHUNK = _CHUNK_IMG_ROWS * _W
_NCHUNK = _NPIX // _CHUNK
_RWIN = 24.0
_INV2S2 = 1.0 / 32.0


def _hist_body(pred_hbm, kp_hbm, cnt_hbm, buf0, buf1, cnt_v, kp_v, sem0, sem1):
    wid = lax.axis_index("s") * 2 + lax.axis_index("c")

    pltpu.sync_copy(kp_hbm, kp_v)

    ones16 = jnp.ones((16,), jnp.float32)

    def accumulate(loss):
        bits = lax.bitcast_convert_type(loss, jnp.int32)
        bins = lax.shift_right_logical(bits, _SHIFT)
        plsc.addupdate_scatter(cnt_v, [bins], ones16)

    def chunk_src(r, cc):
        return pred_hbm.at[pl.ds(r * _NPIX + cc * _CHUNK, _CHUNK)]

    def process_row(r):
        def zbody(i, carry):
            z = jnp.zeros((16,), jnp.float32)
            for u in range(4):
                cnt_v[pl.ds(i * 64 + u * 16, 16)] = z
            return carry
        lax.fori_loop(0, _NBINS // 64, zbody, 0)

        kvec = kp_v[pl.ds(r * 16, 16)]
        kx = kvec[0]
        ky = kvec[1]
        kyi = ky.astype(jnp.int32)
        wlo = jnp.clip(kyi - 26, 0, _H)
        whi = jnp.clip(kyi + 27, 0, _H)
        wstart = wlo * _W
        wend = whi * _W

        def pure_range(buf, lo, hi):
            def body(i, carry):
                base = lo + i * 64
                for u in range(4):
                    v = buf[pl.ds(base + u * 16, 16)]
                    accumulate(v * v)
                return carry
            lax.fori_loop(0, (hi - lo) // 64, body, 0)

        def win_range(buf, lo, hi, a):
            def wbody(iy, carry):
                off = lo + iy * _W
                y = lax.shift_right_logical(a + off, 9)
                dy = y.astype(jnp.float32) - ky
                dy2 = dy * dy

                def jb(j, c2):
                    v = buf[pl.ds(off + j * 16, 16)]
                    xv = (lax.iota(jnp.int32, 16) + j * 16).astype(jnp.float32)
                    dx = xv - kx
                    gt = jnp.exp((dx * dx + dy2) * (-_INV2S2))
                    dlt = v - gt
                    accumulate(dlt * dlt)
                    return c2
                lax.fori_loop(0, _W // 16, jb, 0)
                return carry
            lax.fori_loop(0, (hi - lo) // _W, wbody, 0)

        def process_chunk(buf, cc):
            a = cc * _CHUNK
            p1 = jnp.clip(wstart, a, a + _CHUNK) - a
            p2 = jnp.clip(wend, a, a + _CHUNK) - a
            pure_range(buf, 0, p1)
            win_range(buf, p1, p2, a)
            pure_range(buf, p2, _CHUNK)

        pltpu.async_copy(chunk_src(r, 0), buf0, sem0)

        def hbody(h, carry):
            c0 = 2 * h
            pltpu.async_copy(chunk_src(r, c0 + 1), buf1, sem1)
            pltpu.make_async_copy(chunk_src(r, c0), buf0, sem0).wait()
            process_chunk(buf0, c0)

            @pl.when(h < _NCHUNK // 2 - 1)
            def _():
                pltpu.async_copy(chunk_src(r, c0 + 2), buf0, sem0)

            pltpu.make_async_copy(chunk_src(r, c0 + 1), buf1, sem1).wait()
            process_chunk(buf1, c0 + 1)
            return carry
        lax.fori_loop(0, _NCHUNK // 2, hbody, 0)

        pltpu.sync_copy(cnt_v, cnt_hbm.at[r])

    nrows = jnp.where(wid < _ROWS - 64, 3, 2)

    def row_body(i, carry):
        process_row(wid + 32 * i)
        return carry
    lax.fori_loop(0, nrows, row_body, 0)


@functools.cache
def _hist():
    return pl.kernel(
        _hist_body,
        out_type=jax.ShapeDtypeStruct((_PAD_ROWS, _NBINS), jnp.float32),
        mesh=plsc.VectorSubcoreMesh(
            core_axis_name="c", subcore_axis_name="s", num_cores=2, num_subcores=16
        ),
        scratch_types=[
            pltpu.VMEM((_CHUNK,), jnp.float32),
            pltpu.VMEM((_CHUNK,), jnp.float32),
            pltpu.VMEM((_NBINS,), jnp.float32),
            pltpu.VMEM((16 * _ROWS,), jnp.float32),
            pltpu.SemaphoreType.DMA,
            pltpu.SemaphoreType.DMA,
        ],
        compiler_params=pltpu.CompilerParams(needs_layout_passes=False),
    )


def _select_body(cnt_ref, out_ref):
    cnt = cnt_ref[...]

    bii = (lax.broadcasted_iota(jnp.int32, (_PAD_ROWS, 128, 128), 1) * 128
           + lax.broadcasted_iota(jnp.int32, (_PAD_ROWS, 128, 128), 2))
    val = lax.bitcast_convert_type(
        lax.shift_left(bii, _SHIFT) + (1 << (_SHIFT - 1)), jnp.float32)
    val = jnp.where(bii >= (0x7F800000 >> _SHIFT), 0.0, val)
    sm = cnt * val

    i2 = lax.broadcasted_iota(jnp.int32, (128, 128), 0)
    j2 = lax.broadcasted_iota(jnp.int32, (128, 128), 1)
    t_incl = (i2 <= j2).astype(jnp.float32)
    t_strict = (i2 < j2).astype(jnp.float32)

    dn3 = (((2,), (0,)), ((), ()))
    fine_c = lax.dot_general(cnt, t_incl, dn3, preferred_element_type=jnp.float32)
    fine_s = lax.dot_general(sm, t_incl, dn3, preferred_element_type=jnp.float32)

    cs_c = jnp.sum(cnt, axis=2)
    cs_s = jnp.sum(sm, axis=2)
    dn2 = (((1,), (0,)), ((), ()))
    coarse_c = lax.dot_general(cs_c, t_strict, dn2, preferred_element_type=jnp.float32)
    coarse_s = lax.dot_general(cs_s, t_strict, dn2, preferred_element_type=jnp.float32)

    cum_c = coarse_c[:, :, None] + fine_c
    cum_s = coarse_s[:, :, None] + fine_s

    bi = bii.astype(jnp.float32)
    crossed = cum_c > float(_NPIX - _KEEP)
    bstar = jnp.min(jnp.where(crossed, bi, jnp.float32(3.0e7)), axis=2)
    bstar = jnp.min(bstar, axis=1, keepdims=True)
    sel = (bi == bstar[:, :, None]).astype(jnp.float32)

    def pick(x):
        return jnp.sum(jnp.sum(sel * x, axis=2), axis=1, keepdims=True)

    cum_c_b = pick(cum_c)
    cum_s_b = pick(cum_s)
    cnt_b = pick(cnt)
    sum_b = pick(sm)
    s_tot = jnp.sum(jnp.sum(sm, axis=2), axis=1, keepdims=True)

    need = float(_KEEP - _NPIX) + cum_c_b
    est = sum_b / jnp.maximum(cnt_b, 1.0)
    row_sum = (s_tot - cum_s_b) + need * est

    valid = lax.broadcasted_iota(jnp.int32, (_PAD_ROWS, 1), 0) < _ROWS
    total = jnp.sum(jnp.where(valid, row_sum, 0.0))
    out_ref[...] = jnp.broadcast_to(total / float(_ROWS * _KEEP), (1, 1))


_select = pl.pallas_call(
    _select_body,
    out_shape=jax.ShapeDtypeStruct((1, 1), jnp.float32),
)


def kernel(pred_heatmap, gt_keypoints):
    pred_flat = pred_heatmap.reshape(-1)
    kp_pad = jnp.zeros((_ROWS, 16), jnp.float32)
    kp_flat = kp_pad.at[:, :2].set(gt_keypoints.reshape(_ROWS, 2)).reshape(-1)
    cnt = _hist()(pred_flat, kp_flat)
    out = _select(cnt.reshape(_PAD_ROWS, 128, 128))
    return out.reshape(())


# speedup vs baseline: 139.3730x; 2.5640x over previous
import functools

import jax
import jax.numpy as jnp
from jax import lax
from jax.experimental import pallas as pl
from jax.experimental.pallas import tpu as pltpu
from jax.experimental.pallas import tpu_sc as plsc

_B, _K, _H, _W = 4, 17, 512, 512
_ROWS = _B * _K
_NPIX = _H * _W
_KEEP = int(_NPIX * 0.2)
_NBINS = 16384
_SHIFT = 17
_PAD_ROWS = 72
_NW = 32
_CHUNK_IMG_ROWS = 64
_CHUNK = _CHUNK_IMG_ROWS * _W
_NCHUNK = _NPIX // _CHUNK
_RWIN = 24.0
_INV2S2 = 1.0 / 32.0


def _hist_body(pred_hbm, kp_hbm, cnt_hbm, buf0, buf1, cnt_v, kp_v, sem0, sem1):
    wid = lax.axis_index("s") * 2 + lax.axis_index("c")

    pltpu.sync_copy(kp_hbm, kp_v)

    ones16 = jnp.ones((16,), jnp.float32)

    def accumulate(loss):
        bits = lax.bitcast_convert_type(loss, jnp.int32)
        bins = lax.shift_right_logical(bits, _SHIFT)
        plsc.addupdate_scatter(cnt_v, [bins], ones16)

    def chunk_src(r, cc):
        return pred_hbm.at[pl.ds(r * _NPIX + cc * _CHUNK, _CHUNK)]

    def process_row(r):
        @plsc.parallel_loop(0, _NBINS, step=16, unroll=8)
        def _(off):
            cnt_v[pl.ds(off, 16)] = jnp.zeros((16,), jnp.float32)

        kvec = kp_v[pl.ds(r * 16, 16)]
        kx = kvec[0]
        ky = kvec[1]
        kyi = ky.astype(jnp.int32)
        wlo = jnp.clip(kyi - 26, 0, _H)
        whi = jnp.clip(kyi + 27, 0, _H)
        wstart = wlo * _W
        wend = whi * _W

        def pure_range(buf, lo, hi):
            @plsc.parallel_loop(lo, hi, step=16, unroll=8)
            def _(off):
                v = buf[pl.ds(off, 16)]
                accumulate(v * v)

        def win_range(buf, lo, hi, a):
            def wbody(iy, carry):
                off = lo + iy * _W
                y = lax.shift_right_logical(a + off, 9)
                dy = y.astype(jnp.float32) - ky
                dy2 = dy * dy

                @plsc.parallel_loop(0, _W, step=16, unroll=4)
                def _(x0):
                    v = buf[pl.ds(off + x0, 16)]
                    xv = (lax.iota(jnp.int32, 16) + x0).astype(jnp.float32)
                    dx = xv - kx
                    gt = jnp.exp((dx * dx + dy2) * (-_INV2S2))
                    dlt = v - gt
                    accumulate(dlt * dlt)
                return carry
            lax.fori_loop(0, (hi - lo) // _W, wbody, 0)

        def process_chunk(buf, cc):
            a = cc * _CHUNK
            p1 = jnp.clip(wstart, a, a + _CHUNK) - a
            p2 = jnp.clip(wend, a, a + _CHUNK) - a
            pure_range(buf, 0, p1)
            win_range(buf, p1, p2, a)
            pure_range(buf, p2, _CHUNK)

        pltpu.async_copy(chunk_src(r, 0), buf0, sem0)

        def hbody(h, carry):
            c0 = 2 * h
            pltpu.async_copy(chunk_src(r, c0 + 1), buf1, sem1)
            pltpu.make_async_copy(chunk_src(r, c0), buf0, sem0).wait()
            process_chunk(buf0, c0)

            @pl.when(h < _NCHUNK // 2 - 1)
            def _():
                pltpu.async_copy(chunk_src(r, c0 + 2), buf0, sem0)

            pltpu.make_async_copy(chunk_src(r, c0 + 1), buf1, sem1).wait()
            process_chunk(buf1, c0 + 1)
            return carry
        lax.fori_loop(0, _NCHUNK // 2, hbody, 0)

        pltpu.sync_copy(cnt_v, cnt_hbm.at[r])

    nrows = jnp.where(wid < _ROWS - 64, 3, 2)

    def row_body(i, carry):
        process_row(wid + 32 * i)
        return carry
    lax.fori_loop(0, nrows, row_body, 0)


@functools.cache
def _hist():
    return pl.kernel(
        _hist_body,
        out_type=jax.ShapeDtypeStruct((_PAD_ROWS, _NBINS), jnp.float32),
        mesh=plsc.VectorSubcoreMesh(
            core_axis_name="c", subcore_axis_name="s", num_cores=2, num_subcores=16
        ),
        scratch_types=[
            pltpu.VMEM((_CHUNK,), jnp.float32),
            pltpu.VMEM((_CHUNK,), jnp.float32),
            pltpu.VMEM((_NBINS,), jnp.float32),
            pltpu.VMEM((16 * _ROWS,), jnp.float32),
            pltpu.SemaphoreType.DMA,
            pltpu.SemaphoreType.DMA,
        ],
        compiler_params=pltpu.CompilerParams(needs_layout_passes=False),
    )


def _select_body(cnt_ref, out_ref):
    cnt = cnt_ref[...]

    bii = (lax.broadcasted_iota(jnp.int32, (_PAD_ROWS, 128, 128), 1) * 128
           + lax.broadcasted_iota(jnp.int32, (_PAD_ROWS, 128, 128), 2))
    val = lax.bitcast_convert_type(
        lax.shift_left(bii, _SHIFT) + (1 << (_SHIFT - 1)), jnp.float32)
    val = jnp.where(bii >= (0x7F800000 >> _SHIFT), 0.0, val)
    sm = cnt * val

    i2 = lax.broadcasted_iota(jnp.int32, (128, 128), 0)
    j2 = lax.broadcasted_iota(jnp.int32, (128, 128), 1)
    t_incl = (i2 <= j2).astype(jnp.float32)
    t_strict = (i2 < j2).astype(jnp.float32)

    dn3 = (((2,), (0,)), ((), ()))
    fine_c = lax.dot_general(cnt, t_incl, dn3, preferred_element_type=jnp.float32)
    fine_s = lax.dot_general(sm, t_incl, dn3, preferred_element_type=jnp.float32)

    cs_c = jnp.sum(cnt, axis=2)
    cs_s = jnp.sum(sm, axis=2)
    dn2 = (((1,), (0,)), ((), ()))
    coarse_c = lax.dot_general(cs_c, t_strict, dn2, preferred_element_type=jnp.float32)
    coarse_s = lax.dot_general(cs_s, t_strict, dn2, preferred_element_type=jnp.float32)

    cum_c = coarse_c[:, :, None] + fine_c
    cum_s = coarse_s[:, :, None] + fine_s

    bi = bii.astype(jnp.float32)
    crossed = cum_c > float(_NPIX - _KEEP)
    bstar = jnp.min(jnp.where(crossed, bi, jnp.float32(3.0e7)), axis=2)
    bstar = jnp.min(bstar, axis=1, keepdims=True)
    sel = (bi == bstar[:, :, None]).astype(jnp.float32)

    def pick(x):
        return jnp.sum(jnp.sum(sel * x, axis=2), axis=1, keepdims=True)

    cum_c_b = pick(cum_c)
    cum_s_b = pick(cum_s)
    cnt_b = pick(cnt)
    sum_b = pick(sm)
    s_tot = jnp.sum(jnp.sum(sm, axis=2), axis=1, keepdims=True)

    need = float(_KEEP - _NPIX) + cum_c_b
    est = sum_b / jnp.maximum(cnt_b, 1.0)
    row_sum = (s_tot - cum_s_b) + need * est

    valid = lax.broadcasted_iota(jnp.int32, (_PAD_ROWS, 1), 0) < _ROWS
    total = jnp.sum(jnp.where(valid, row_sum, 0.0))
    out_ref[...] = jnp.broadcast_to(total / float(_ROWS * _KEEP), (1, 1))


_select = pl.pallas_call(
    _select_body,
    out_shape=jax.ShapeDtypeStruct((1, 1), jnp.float32),
)


def kernel(pred_heatmap, gt_keypoints):
    pred_flat = pred_heatmap.reshape(-1)
    kp_pad = jnp.zeros((_ROWS, 16), jnp.float32)
    kp_flat = kp_pad.at[:, :2].set(gt_keypoints.reshape(_ROWS, 2)).reshape(-1)
    cnt = _hist()(pred_flat, kp_flat)
    out = _select(cnt.reshape(_PAD_ROWS, 128, 128))
    return out.reshape(())


# speedup vs baseline: 186.3385x; 1.3370x over previous
import functools

import jax
import jax.numpy as jnp
from jax import lax
from jax.experimental import pallas as pl
from jax.experimental.pallas import tpu as pltpu
from jax.experimental.pallas import tpu_sc as plsc

_B, _K, _H, _W = 4, 17, 512, 512
_ROWS = _B * _K
_NPIX = _H * _W
_KEEP = int(_NPIX * 0.2)
_NBINS = 16384
_SHIFT = 17
_PAD_ROWS = 72
_NW = 32
_CHUNK_IMG_ROWS = 64
_CHUNK = _CHUNK_IMG_ROWS * _W
_NCHUNK = _NPIX // _CHUNK
_RWIN = 24.0
_INV2S2 = 1.0 / 32.0


def _hist_body(pred_hbm, kp_hbm, cnt_hbm, buf0, buf1, cnt_v, kp_v, sem0, sem1):
    wid = lax.axis_index("s") * 2 + lax.axis_index("c")

    pltpu.sync_copy(kp_hbm, kp_v)

    ones16 = jnp.ones((16,), jnp.float32)

    def accumulate(loss):
        bits = lax.bitcast_convert_type(loss, jnp.int32)
        bins = lax.shift_right_logical(bits, _SHIFT)
        plsc.addupdate_scatter(cnt_v, [bins], ones16)

    def chunk_src(r, cc):
        return pred_hbm.at[r, pl.ds(cc * _CHUNK_IMG_ROWS, _CHUNK_IMG_ROWS)]

    def process_row(r):
        @plsc.parallel_loop(0, _NBINS, step=16, unroll=8)
        def _(off):
            cnt_v[pl.ds(off, 16)] = jnp.zeros((16,), jnp.float32)

        kvec = kp_v[pl.ds(r * 16, 16)]
        kx = kvec[0]
        ky = kvec[1]
        kyi = ky.astype(jnp.int32)
        wlo = jnp.clip(kyi - 26, 0, _H)
        whi = jnp.clip(kyi + 27, 0, _H)
        wstart = wlo * _W
        wend = whi * _W

        def pure_range(buf, lo, hi):
            @plsc.parallel_loop(lo, hi, step=16, unroll=8)
            def _(off):
                yy = lax.shift_right_logical(off, 9)
                x0 = jnp.bitwise_and(off, _W - 1)
                v = buf[yy, pl.ds(x0, 16)]
                accumulate(v * v)

        def win_range(buf, lo, hi, a):
            def wbody(iy, carry):
                off = lo + iy * _W
                y = lax.shift_right_logical(a + off, 9)
                dy = y.astype(jnp.float32) - ky
                dy2 = dy * dy

                yloc = lax.shift_right_logical(off, 9)

                @plsc.parallel_loop(0, _W, step=16, unroll=4)
                def _(x0):
                    v = buf[yloc, pl.ds(x0, 16)]
                    xv = (lax.iota(jnp.int32, 16) + x0).astype(jnp.float32)
                    dx = xv - kx
                    gt = jnp.exp((dx * dx + dy2) * (-_INV2S2))
                    dlt = v - gt
                    accumulate(dlt * dlt)
                return carry
            lax.fori_loop(0, (hi - lo) // _W, wbody, 0)

        def process_chunk(buf, cc):
            a = cc * _CHUNK
            p1 = jnp.clip(wstart, a, a + _CHUNK) - a
            p2 = jnp.clip(wend, a, a + _CHUNK) - a
            pure_range(buf, 0, p1)
            win_range(buf, p1, p2, a)
            pure_range(buf, p2, _CHUNK)

        pltpu.async_copy(chunk_src(r, 0), buf0, sem0)

        def hbody(h, carry):
            c0 = 2 * h
            pltpu.async_copy(chunk_src(r, c0 + 1), buf1, sem1)
            pltpu.make_async_copy(chunk_src(r, c0), buf0, sem0).wait()
            process_chunk(buf0, c0)

            @pl.when(h < _NCHUNK // 2 - 1)
            def _():
                pltpu.async_copy(chunk_src(r, c0 + 2), buf0, sem0)

            pltpu.make_async_copy(chunk_src(r, c0 + 1), buf1, sem1).wait()
            process_chunk(buf1, c0 + 1)
            return carry
        lax.fori_loop(0, _NCHUNK // 2, hbody, 0)

        pltpu.sync_copy(cnt_v, cnt_hbm.at[r])

    nrows = jnp.where(wid < _ROWS - 64, 3, 2)

    def row_body(i, carry):
        process_row(wid + 32 * i)
        return carry
    lax.fori_loop(0, nrows, row_body, 0)


@functools.cache
def _hist():
    return pl.kernel(
        _hist_body,
        out_type=jax.ShapeDtypeStruct((_PAD_ROWS, _NBINS), jnp.float32),
        mesh=plsc.VectorSubcoreMesh(
            core_axis_name="c", subcore_axis_name="s", num_cores=2, num_subcores=16
        ),
        scratch_types=[
            pltpu.VMEM((_CHUNK_IMG_ROWS, _W), jnp.float32),
            pltpu.VMEM((_CHUNK_IMG_ROWS, _W), jnp.float32),
            pltpu.VMEM((_NBINS,), jnp.float32),
            pltpu.VMEM((16 * _ROWS,), jnp.float32),
            pltpu.SemaphoreType.DMA,
            pltpu.SemaphoreType.DMA,
        ],
        compiler_params=pltpu.CompilerParams(
            needs_layout_passes=False, use_tc_tiling_on_sc=True
        ),
    )


def _select_body(cnt_ref, out_ref):
    cnt = cnt_ref[...]

    bii = (lax.broadcasted_iota(jnp.int32, (_PAD_ROWS, 128, 128), 1) * 128
           + lax.broadcasted_iota(jnp.int32, (_PAD_ROWS, 128, 128), 2))
    val = lax.bitcast_convert_type(
        lax.shift_left(bii, _SHIFT) + (1 << (_SHIFT - 1)), jnp.float32)
    val = jnp.where(bii >= (0x7F800000 >> _SHIFT), 0.0, val)
    sm = cnt * val

    i2 = lax.broadcasted_iota(jnp.int32, (128, 128), 0)
    j2 = lax.broadcasted_iota(jnp.int32, (128, 128), 1)
    t_incl = (i2 <= j2).astype(jnp.float32)
    t_strict = (i2 < j2).astype(jnp.float32)

    dn3 = (((2,), (0,)), ((), ()))
    fine_c = lax.dot_general(cnt, t_incl, dn3, preferred_element_type=jnp.float32)
    fine_s = lax.dot_general(sm, t_incl, dn3, preferred_element_type=jnp.float32)

    cs_c = jnp.sum(cnt, axis=2)
    cs_s = jnp.sum(sm, axis=2)
    dn2 = (((1,), (0,)), ((), ()))
    coarse_c = lax.dot_general(cs_c, t_strict, dn2, preferred_element_type=jnp.float32)
    coarse_s = lax.dot_general(cs_s, t_strict, dn2, preferred_element_type=jnp.float32)

    cum_c = coarse_c[:, :, None] + fine_c
    cum_s = coarse_s[:, :, None] + fine_s

    bi = bii.astype(jnp.float32)
    crossed = cum_c > float(_NPIX - _KEEP)
    bstar = jnp.min(jnp.where(crossed, bi, jnp.float32(3.0e7)), axis=2)
    bstar = jnp.min(bstar, axis=1, keepdims=True)
    sel = (bi == bstar[:, :, None]).astype(jnp.float32)

    def pick(x):
        return jnp.sum(jnp.sum(sel * x, axis=2), axis=1, keepdims=True)

    cum_c_b = pick(cum_c)
    cum_s_b = pick(cum_s)
    cnt_b = pick(cnt)
    sum_b = pick(sm)
    s_tot = jnp.sum(jnp.sum(sm, axis=2), axis=1, keepdims=True)

    need = float(_KEEP - _NPIX) + cum_c_b
    est = sum_b / jnp.maximum(cnt_b, 1.0)
    row_sum = (s_tot - cum_s_b) + need * est

    valid = lax.broadcasted_iota(jnp.int32, (_PAD_ROWS, 1), 0) < _ROWS
    total = jnp.sum(jnp.where(valid, row_sum, 0.0))
    out_ref[...] = jnp.broadcast_to(total / float(_ROWS * _KEEP), (1, 1))


_select = pl.pallas_call(
    _select_body,
    out_shape=jax.ShapeDtypeStruct((1, 1), jnp.float32),
)


def kernel(pred_heatmap, gt_keypoints):
    pred_flat = pred_heatmap.reshape(_ROWS, _H, _W)
    kp_pad = jnp.zeros((_ROWS, 16), jnp.float32)
    kp_flat = kp_pad.at[:, :2].set(gt_keypoints.reshape(_ROWS, 2)).reshape(-1)
    cnt = _hist()(pred_flat, kp_flat)
    out = _select(cnt.reshape(_PAD_ROWS, 128, 128))
    return out.reshape(())


# speedup vs baseline: 224.4772x; 1.2047x over previous
import functools

import jax
import jax.numpy as jnp
from jax import lax
from jax.experimental import pallas as pl
from jax.experimental.pallas import tpu as pltpu
from jax.experimental.pallas import tpu_sc as plsc

_B, _K, _H, _W = 4, 17, 512, 512
_ROWS = _B * _K
_NPIX = _H * _W
_KEEP = int(_NPIX * 0.2)
_NBINS = 16384
_SHIFT = 17
_OUT_ROWS = 96
_NW = 32
_CHUNK_IMG_ROWS = 64
_CHUNK = _CHUNK_IMG_ROWS * _W
_NCHUNK = _NPIX // _CHUNK
_RWIN = 24.0
_INV2S2 = 1.0 / 32.0


def _hist_body(pred_hbm, kp_hbm, cnt_hbm, buf0, buf1, cnt_v, kp_v, sem0, sem1):
    wid = lax.axis_index("s") * 2 + lax.axis_index("c")

    pltpu.sync_copy(kp_hbm, kp_v)

    ones16 = jnp.ones((16,), jnp.float32)

    def accumulate(loss):
        bits = lax.bitcast_convert_type(loss, jnp.int32)
        bins = lax.shift_right_logical(bits, _SHIFT)
        plsc.addupdate_scatter(cnt_v, [bins], ones16)

    def chunk_src(r, cc):
        return pred_hbm.at[r, pl.ds(cc * _CHUNK_IMG_ROWS, _CHUNK_IMG_ROWS)]

    def zero_hist():
        @plsc.parallel_loop(0, _NBINS, step=16, unroll=8)
        def _(off):
            cnt_v[pl.ds(off, 16)] = jnp.zeros((16,), jnp.float32)

    def row_ctx(r):
        kvec = kp_v[pl.ds(r * 16, 16)]
        kx = kvec[0]
        ky = kvec[1]
        kyi = ky.astype(jnp.int32)
        wlo = jnp.clip(kyi - 26, 0, _H)
        whi = jnp.clip(kyi + 27, 0, _H)
        return kx, ky, wlo * _W, whi * _W

    def pure_range(buf, lo, hi):
        @plsc.parallel_loop(lo, hi, step=16, unroll=8)
        def _(off):
            yy = lax.shift_right_logical(off, 9)
            x0 = jnp.bitwise_and(off, _W - 1)
            v = buf[yy, pl.ds(x0, 16)]
            accumulate(v * v)

    def win_range(buf, lo, hi, a, kx, ky):
        def wbody(iy, carry):
            off = lo + iy * _W
            y = lax.shift_right_logical(a + off, 9)
            dy = y.astype(jnp.float32) - ky
            dy2 = dy * dy

            yloc = lax.shift_right_logical(off, 9)

            @plsc.parallel_loop(0, _W, step=16, unroll=4)
            def _(x0):
                v = buf[yloc, pl.ds(x0, 16)]
                xv = (lax.iota(jnp.int32, 16) + x0).astype(jnp.float32)
                dx = xv - kx
                gt = jnp.exp((dx * dx + dy2) * (-_INV2S2))
                dlt = v - gt
                accumulate(dlt * dlt)
            return carry
        lax.fori_loop(0, (hi - lo) // _W, wbody, 0)

    def process_chunk(buf, cc, ctx):
        kx, ky, wstart, wend = ctx
        a = cc * _CHUNK
        p1 = jnp.clip(wstart, a, a + _CHUNK) - a
        p2 = jnp.clip(wend, a, a + _CHUNK) - a
        pure_range(buf, 0, p1)
        win_range(buf, p1, p2, a, kx, ky)
        pure_range(buf, p2, _CHUNK)

    def process_row(r):
        zero_hist()
        ctx = row_ctx(r)

        pltpu.async_copy(chunk_src(r, 0), buf0, sem0)

        def hbody(h, carry):
            c0 = 2 * h
            pltpu.async_copy(chunk_src(r, c0 + 1), buf1, sem1)
            pltpu.make_async_copy(chunk_src(r, c0), buf0, sem0).wait()
            process_chunk(buf0, c0, ctx)

            @pl.when(h < _NCHUNK // 2 - 1)
            def _():
                pltpu.async_copy(chunk_src(r, c0 + 2), buf0, sem0)

            pltpu.make_async_copy(chunk_src(r, c0 + 1), buf1, sem1).wait()
            process_chunk(buf1, c0 + 1, ctx)
            return carry
        lax.fori_loop(0, _NCHUNK // 2, hbody, 0)

        pltpu.sync_copy(cnt_v, cnt_hbm.at[r])

    def row_body(i, carry):
        process_row(wid + 32 * i)
        return carry
    lax.fori_loop(0, 2, row_body, 0)

    r8 = 64 + lax.shift_right_logical(wid, 3)
    s8 = jnp.bitwise_and(wid, 7)
    pltpu.async_copy(chunk_src(r8, s8), buf0, sem0)
    zero_hist()
    ctx8 = row_ctx(r8)
    pltpu.make_async_copy(chunk_src(r8, s8), buf0, sem0).wait()
    process_chunk(buf0, s8, ctx8)
    pltpu.sync_copy(cnt_v, cnt_hbm.at[64 + wid])


@functools.cache
def _hist():
    return pl.kernel(
        _hist_body,
        out_type=jax.ShapeDtypeStruct((_OUT_ROWS, _NBINS), jnp.float32),
        mesh=plsc.VectorSubcoreMesh(
            core_axis_name="c", subcore_axis_name="s", num_cores=2, num_subcores=16
        ),
        scratch_types=[
            pltpu.VMEM((_CHUNK_IMG_ROWS, _W), jnp.float32),
            pltpu.VMEM((_CHUNK_IMG_ROWS, _W), jnp.float32),
            pltpu.VMEM((_NBINS,), jnp.float32),
            pltpu.VMEM((16 * _ROWS,), jnp.float32),
            pltpu.SemaphoreType.DMA,
            pltpu.SemaphoreType.DMA,
        ],
        compiler_params=pltpu.CompilerParams(
            needs_layout_passes=False, use_tc_tiling_on_sc=True
        ),
    )


def _select_body(cnt_ref, out_ref):
    x = cnt_ref[...]
    head = x[0:64]
    tail = jnp.sum(x[64:96].reshape(4, 8, 128, 128), axis=1)
    cnt = jnp.concatenate([head, tail], axis=0)

    bii = (lax.broadcasted_iota(jnp.int32, (_ROWS, 128, 128), 1) * 128
           + lax.broadcasted_iota(jnp.int32, (_ROWS, 128, 128), 2))
    val = lax.bitcast_convert_type(
        lax.shift_left(bii, _SHIFT) + (1 << (_SHIFT - 1)), jnp.float32)
    val = jnp.where(bii >= (0x7F800000 >> _SHIFT), 0.0, val)
    sm = cnt * val

    i2 = lax.broadcasted_iota(jnp.int32, (128, 128), 0)
    j2 = lax.broadcasted_iota(jnp.int32, (128, 128), 1)
    t_incl = (i2 <= j2).astype(jnp.float32)
    t_strict = (i2 < j2).astype(jnp.float32)

    dn3 = (((2,), (0,)), ((), ()))
    fine_c = lax.dot_general(cnt, t_incl, dn3, preferred_element_type=jnp.float32)
    fine_s = lax.dot_general(sm, t_incl, dn3, preferred_element_type=jnp.float32)

    cs_c = jnp.sum(cnt, axis=2)
    cs_s = jnp.sum(sm, axis=2)
    dn2 = (((1,), (0,)), ((), ()))
    coarse_c = lax.dot_general(cs_c, t_strict, dn2, preferred_element_type=jnp.float32)
    coarse_s = lax.dot_general(cs_s, t_strict, dn2, preferred_element_type=jnp.float32)

    cum_c = coarse_c[:, :, None] + fine_c
    cum_s = coarse_s[:, :, None] + fine_s

    bi = bii.astype(jnp.float32)
    crossed = cum_c > float(_NPIX - _KEEP)
    bstar = jnp.min(jnp.where(crossed, bi, jnp.float32(3.0e7)), axis=2)
    bstar = jnp.min(bstar, axis=1, keepdims=True)
    sel = (bi == bstar[:, :, None]).astype(jnp.float32)

    def pick(x):
        return jnp.sum(jnp.sum(sel * x, axis=2), axis=1, keepdims=True)

    cum_c_b = pick(cum_c)
    cum_s_b = pick(cum_s)
    cnt_b = pick(cnt)
    sum_b = pick(sm)
    s_tot = jnp.sum(jnp.sum(sm, axis=2), axis=1, keepdims=True)

    need = float(_KEEP - _NPIX) + cum_c_b
    est = sum_b / jnp.maximum(cnt_b, 1.0)
    row_sum = (s_tot - cum_s_b) + need * est

    total = jnp.sum(row_sum)
    out_ref[...] = jnp.broadcast_to(total / float(_ROWS * _KEEP), (1, 1))


_select = pl.pallas_call(
    _select_body,
    out_shape=jax.ShapeDtypeStruct((1, 1), jnp.float32),
)


def kernel(pred_heatmap, gt_keypoints):
    pred_flat = pred_heatmap.reshape(_ROWS, _H, _W)
    kp_pad = jnp.zeros((_ROWS, 16), jnp.float32)
    kp_flat = kp_pad.at[:, :2].set(gt_keypoints.reshape(_ROWS, 2)).reshape(-1)
    cnt = _hist()(pred_flat, kp_flat)
    out = _select(cnt.reshape(_OUT_ROWS, 128, 128))
    return out.reshape(())
